# Initial kernel scaffold; baseline (speedup 1.0000x reference)
import jax
import jax.numpy as jnp
from jax.experimental import pallas as pl


def kernel(x, edge_index, Wskip, bskip, Wl1, bl1, Wr1, Wl2, bl2, Wr2, Wl3, bl3, Wr3, a):
    raise NotImplementedError("write your pallas kernel here")


# speedup vs baseline: 3.5279x; 3.5279x over previous
import jax
import jax.numpy as jnp
from jax import lax
from jax.experimental import pallas as pl
from jax.experimental.pallas import tpu as pltpu
from jax.experimental.pallas import tpu_sc as plsc

_N = 10000
_E = 320000
_D = 128
_NC = 2
_NS = 16
_NW = _NC * _NS
_CH = 128
_CPT = 79
_EPAD = _NW * _CPT * _CH
_TRASH = _N
_ROWS = 10240
_RPT = _ROWS // _NS


def _mesh():
  return plsc.VectorSubcoreMesh(core_axis_name="c", subcore_axis_name="s",
                                num_cores=_NC, num_subcores=_NS)


def _seg_sum_body(feat, srcp, dstp, zrows, out,
                  src_v, dst_v, rows_v, acc, sem):
  cid = lax.axis_index("c")
  sid = lax.axis_index("s")
  wid = sid * _NC + cid

  pltpu.sync_copy(zrows, rows_v)
  r0 = sid * _RPT
  for k in range(_RPT // _CH):
    pltpu.sync_copy(rows_v, acc.at[pl.ds(r0 + k * _CH, _CH)])
  plsc.subcore_barrier()

  base = wid * _CPT * _CH

  def step(j, carry):
    e0 = pl.multiple_of(base + j * _CH, _CH)
    pltpu.sync_copy(srcp.at[pl.ds(e0, _CH)], src_v)
    pltpu.sync_copy(dstp.at[pl.ds(e0, _CH)], dst_v)
    pltpu.async_copy(feat.at[src_v], rows_v, sem).wait()
    pltpu.sync_copy(rows_v, acc.at[dst_v], add=True)
    return carry

  lax.fori_loop(0, _CPT, step, 0)
  plsc.subcore_barrier()

  pltpu.sync_copy(acc.at[pl.ds(r0, _RPT)], out.at[cid, pl.ds(r0, _RPT)])


_seg_sum = pl.kernel(
    _seg_sum_body,
    out_type=jax.ShapeDtypeStruct((_NC, _ROWS, _D), jnp.float32),
    mesh=_mesh(),
    scratch_types=[
        pltpu.VMEM((_CH,), jnp.int32),
        pltpu.VMEM((_CH,), jnp.int32),
        pltpu.VMEM((_CH, _D), jnp.float32),
        pltpu.VMEM_SHARED((_ROWS, _D), jnp.float32),
        pltpu.SemaphoreType.DMA,
    ],
)


def _seg_cnt_body(dstp, zrows, ones, out,
                  dst_v, ones_v, rows_v, acc, sem):
  cid = lax.axis_index("c")
  sid = lax.axis_index("s")
  wid = sid * _NC + cid

  pltpu.sync_copy(zrows, rows_v)
  pltpu.sync_copy(ones, ones_v)
  r0 = sid * _RPT
  for k in range(_RPT // _CH):
    pltpu.sync_copy(rows_v, acc.at[pl.ds(r0 + k * _CH, _CH)])
  plsc.subcore_barrier()

  base = wid * _CPT * _CH

  def step(j, carry):
    e0 = pl.multiple_of(base + j * _CH, _CH)
    pltpu.sync_copy(dstp.at[pl.ds(e0, _CH)], dst_v)
    pltpu.sync_copy(ones_v, acc.at[dst_v], add=True)
    return carry

  lax.fori_loop(0, _CPT, step, 0)
  plsc.subcore_barrier()

  pltpu.sync_copy(acc.at[pl.ds(r0, _RPT)], out.at[cid, pl.ds(r0, _RPT)])


_seg_cnt = pl.kernel(
    _seg_cnt_body,
    out_type=jax.ShapeDtypeStruct((_NC, _ROWS, _D), jnp.float32),
    mesh=_mesh(),
    scratch_types=[
        pltpu.VMEM((_CH,), jnp.int32),
        pltpu.VMEM((_CH, _D), jnp.float32),
        pltpu.VMEM((_CH, _D), jnp.float32),
        pltpu.VMEM_SHARED((_ROWS, _D), jnp.float32),
        pltpu.SemaphoreType.DMA,
    ],
)


_R = 1000


def _prelu(h, a):
  return jnp.where(h >= 0, h, a * h)


def _mean_from(p_ref, c_ref):
  p = p_ref[0] + p_ref[1]
  c = c_ref[0][:, 0:1] + c_ref[1][:, 0:1]
  return p * (1.0 / jnp.maximum(c, 1.0))


def _mm(u, w):
  return lax.dot_general(u, w, (((1,), (1,)), ((), ())),
                         preferred_element_type=jnp.float32)


def _norm_rows(o):
  nrm = jnp.sqrt(jnp.sum(o * o, axis=1, keepdims=True))
  return o / jnp.maximum(nrm, 1e-12)


def _dense1_body(p_ref, c_ref, g_ref, wl_ref, bl_ref, wr_ref,
                 wsk_ref, bsk_ref, a_ref, g2_ref, sk_ref):
  a = a_ref[0, 0]
  g = g_ref[...]
  out = _mm(_mean_from(p_ref, c_ref), wl_ref[...]) + bl_ref[...] \
      + _mm(g, wr_ref[...])
  h = _prelu(_norm_rows(out), a)
  sk = _mm(g, wsk_ref[...]) + bsk_ref[...]
  g2_ref[...] = h + sk
  sk_ref[...] = sk


def _dense2_body(p_ref, c_ref, g_ref, wl_ref, bl_ref, wr_ref,
                 sk_ref, a_ref, g3_ref):
  a = a_ref[0, 0]
  out = _mm(_mean_from(p_ref, c_ref), wl_ref[...]) + bl_ref[...] \
      + _mm(g_ref[...], wr_ref[...])
  g3_ref[...] = _prelu(_norm_rows(out), a) + sk_ref[...]


def _dense3_body(p_ref, c_ref, g_ref, wl_ref, bl_ref, wr_ref,
                 a_ref, o_ref):
  a = a_ref[0, 0]
  out = _mm(_mean_from(p_ref, c_ref), wl_ref[...]) + bl_ref[...] \
      + _mm(g_ref[...], wr_ref[...])
  o_ref[...] = _prelu(_prelu(_norm_rows(out), a), a)


def _spec_p():
  return pl.BlockSpec((2, _R, _D), lambda i: (0, i, 0))


def _spec_rows():
  return pl.BlockSpec((_R, _D), lambda i: (i, 0))


def _spec_w():
  return pl.BlockSpec((_D, _D), lambda i: (0, 0))


def _spec_b():
  return pl.BlockSpec((1, _D), lambda i: (0, 0))


def _spec_a():
  return pl.BlockSpec((1, 1), lambda i: (0, 0), memory_space=pltpu.SMEM)


_G = (_N // _R,)


_dense1 = pl.pallas_call(
    _dense1_body,
    grid=_G,
    in_specs=[_spec_p(), _spec_p(), _spec_rows(), _spec_w(), _spec_b(),
              _spec_w(), _spec_w(), _spec_b(), _spec_a()],
    out_specs=[_spec_rows(), _spec_rows()],
    out_shape=[jax.ShapeDtypeStruct((_N, _D), jnp.float32),
               jax.ShapeDtypeStruct((_N, _D), jnp.float32)],
)

_dense2 = pl.pallas_call(
    _dense2_body,
    grid=_G,
    in_specs=[_spec_p(), _spec_p(), _spec_rows(), _spec_w(), _spec_b(),
              _spec_w(), _spec_rows(), _spec_a()],
    out_specs=_spec_rows(),
    out_shape=jax.ShapeDtypeStruct((_N, _D), jnp.float32),
)

_dense3 = pl.pallas_call(
    _dense3_body,
    grid=_G,
    in_specs=[_spec_p(), _spec_p(), _spec_rows(), _spec_w(), _spec_b(),
              _spec_w(), _spec_a()],
    out_specs=_spec_rows(),
    out_shape=jax.ShapeDtypeStruct((_N, _D), jnp.float32),
)


def kernel(x, edge_index, Wskip, bskip, Wl1, bl1, Wr1,
           Wl2, bl2, Wr2, Wl3, bl3, Wr3, a):
  src = edge_index[0]
  dst = edge_index[1]
  pad = _EPAD - _E
  srcp = jnp.concatenate([src, jnp.zeros((pad,), jnp.int32)])
  dstp = jnp.concatenate([dst, jnp.full((pad,), _TRASH, jnp.int32)])
  zrows = jnp.zeros((_CH, _D), jnp.float32)
  ones = jnp.ones((_CH, _D), jnp.float32)
  a2 = a.reshape(1, 1)
  bl1r = bl1.reshape(1, _D)
  bl2r = bl2.reshape(1, _D)
  bl3r = bl3.reshape(1, _D)
  bskr = bskip.reshape(1, _D)

  cnt = _seg_cnt(dstp, zrows, ones)
  p1 = _seg_sum(x, srcp, dstp, zrows)
  g2, sk = _dense1(p1, cnt, x, Wl1, bl1r, Wr1, Wskip, bskr, a2)
  p2 = _seg_sum(g2, srcp, dstp, zrows)
  g3 = _dense2(p2, cnt, g2, Wl2, bl2r, Wr2, sk, a2)
  p3 = _seg_sum(g3, srcp, dstp, zrows)
  return _dense3(p3, cnt, g3, Wl3, bl3r, Wr3, a2)


# speedup vs baseline: 4.8629x; 1.3784x over previous
import jax
import jax.numpy as jnp
from jax import lax
from jax.experimental import pallas as pl
from jax.experimental.pallas import tpu as pltpu
from jax.experimental.pallas import tpu_sc as plsc

_N = 10000
_E = 320000
_D = 128
_NC = 2
_NS = 16
_NW = _NC * _NS
_CH = 128
_CPT = 79
_EPAD = _NW * _CPT * _CH
_TRASH = _N
_ROWS = 10240
_RPT = _ROWS // _NS


def _mesh():
  return plsc.VectorSubcoreMesh(core_axis_name="c", subcore_axis_name="s",
                                num_cores=_NC, num_subcores=_NS)


def _seg_sum_body(feat, idx, zrows, out,
                  idx_a, idx_b, rows_a, rows_b, acc, gsa, gsb, isem):
  cid = lax.axis_index("c")
  sid = lax.axis_index("s")
  wid = sid * _NC + cid
  base = wid * _CPT

  pltpu.sync_copy(zrows, rows_a)
  r0 = sid * _RPT
  for k in range(_RPT // _CH):
    pltpu.sync_copy(rows_a, acc.at[pl.ds(r0 + k * _CH, _CH)])

  pltpu.sync_copy(idx.at[base], idx_a)
  pltpu.async_copy(feat.at[idx_a.at[0]], rows_a, gsa)
  pltpu.async_copy(idx.at[base + 1], idx_b, isem)
  plsc.subcore_barrier()

  @pl.loop(0, _CPT - 1, step=2)
  def _(j):
    pltpu.make_async_copy(idx.at[base + j + 1], idx_b, isem).wait()
    pltpu.async_copy(feat.at[idx_b.at[0]], rows_b, gsb)
    pltpu.make_async_copy(feat.at[idx_a.at[0]], rows_a, gsa).wait()
    pltpu.sync_copy(rows_a, acc.at[idx_a.at[1]], add=True)
    pltpu.async_copy(idx.at[base + j + 2], idx_a, isem)

    pltpu.make_async_copy(idx.at[base + j + 2], idx_a, isem).wait()
    pltpu.async_copy(feat.at[idx_a.at[0]], rows_a, gsa)
    pltpu.make_async_copy(feat.at[idx_b.at[0]], rows_b, gsb).wait()
    pltpu.sync_copy(rows_b, acc.at[idx_b.at[1]], add=True)

    @pl.when(j < _CPT - 3)
    def _():
      pltpu.async_copy(idx.at[base + j + 3], idx_b, isem)

  pltpu.make_async_copy(feat.at[idx_a.at[0]], rows_a, gsa).wait()
  pltpu.sync_copy(rows_a, acc.at[idx_a.at[1]], add=True)
  plsc.subcore_barrier()

  pltpu.sync_copy(acc.at[pl.ds(r0, _RPT)], out.at[cid, pl.ds(r0, _RPT)])


_seg_sum = pl.kernel(
    _seg_sum_body,
    out_type=jax.ShapeDtypeStruct((_NC, _ROWS, _D), jnp.float32),
    mesh=_mesh(),
    scratch_types=[
        pltpu.VMEM((2, _CH), jnp.int32),
        pltpu.VMEM((2, _CH), jnp.int32),
        pltpu.VMEM((_CH, _D), jnp.float32),
        pltpu.VMEM((_CH, _D), jnp.float32),
        pltpu.VMEM_SHARED((_ROWS, _D), jnp.float32),
        pltpu.SemaphoreType.DMA,
        pltpu.SemaphoreType.DMA,
        pltpu.SemaphoreType.DMA,
    ],
)


def _seg_cnt_body(idx, zrows, ones, out,
                  idx_all, ones_v, acc, sem):
  cid = lax.axis_index("c")
  sid = lax.axis_index("s")
  wid = sid * _NC + cid
  base = wid * _CPT

  pltpu.sync_copy(zrows, ones_v)
  r0 = sid * _RPT
  for k in range(_RPT // _CH):
    pltpu.sync_copy(ones_v, acc.at[pl.ds(r0 + k * _CH, _CH)])
  pltpu.sync_copy(idx.at[pl.ds(base, _CPT)], idx_all)
  pltpu.sync_copy(ones, ones_v)
  plsc.subcore_barrier()

  @pl.loop(0, _CPT)
  def _(j):
    pltpu.sync_copy(ones_v, acc.at[idx_all.at[j, 1]], add=True)

  plsc.subcore_barrier()
  pltpu.sync_copy(acc.at[pl.ds(r0, _RPT)], out.at[cid, pl.ds(r0, _RPT)])


_seg_cnt = pl.kernel(
    _seg_cnt_body,
    out_type=jax.ShapeDtypeStruct((_NC, _ROWS, _D), jnp.float32),
    mesh=_mesh(),
    scratch_types=[
        pltpu.VMEM((_CPT, 2, _CH), jnp.int32),
        pltpu.VMEM((_CH, _D), jnp.float32),
        pltpu.VMEM_SHARED((_ROWS, _D), jnp.float32),
        pltpu.SemaphoreType.DMA,
    ],
)


_R = 1000


def _prelu(h, a):
  return jnp.where(h >= 0, h, a * h)


def _mean_from(p_ref, c_ref):
  p = p_ref[0] + p_ref[1]
  c = c_ref[0][:, 0:1] + c_ref[1][:, 0:1]
  return p * (1.0 / jnp.maximum(c, 1.0))


def _mm(u, w):
  return lax.dot_general(u, w, (((1,), (1,)), ((), ())),
                         preferred_element_type=jnp.float32)


def _norm_rows(o):
  nrm = jnp.sqrt(jnp.sum(o * o, axis=1, keepdims=True))
  return o / jnp.maximum(nrm, 1e-12)


def _dense1_body(p_ref, c_ref, g_ref, wl_ref, bl_ref, wr_ref,
                 wsk_ref, bsk_ref, a_ref, g2_ref, sk_ref):
  a = a_ref[0, 0]
  g = g_ref[...]
  out = _mm(_mean_from(p_ref, c_ref), wl_ref[...]) + bl_ref[...] \
      + _mm(g, wr_ref[...])
  h = _prelu(_norm_rows(out), a)
  sk = _mm(g, wsk_ref[...]) + bsk_ref[...]
  g2_ref[...] = h + sk
  sk_ref[...] = sk


def _dense2_body(p_ref, c_ref, g_ref, wl_ref, bl_ref, wr_ref,
                 sk_ref, a_ref, g3_ref):
  a = a_ref[0, 0]
  out = _mm(_mean_from(p_ref, c_ref), wl_ref[...]) + bl_ref[...] \
      + _mm(g_ref[...], wr_ref[...])
  g3_ref[...] = _prelu(_norm_rows(out), a) + sk_ref[...]


def _dense3_body(p_ref, c_ref, g_ref, wl_ref, bl_ref, wr_ref,
                 a_ref, o_ref):
  a = a_ref[0, 0]
  out = _mm(_mean_from(p_ref, c_ref), wl_ref[...]) + bl_ref[...] \
      + _mm(g_ref[...], wr_ref[...])
  o_ref[...] = _prelu(_prelu(_norm_rows(out), a), a)


def _spec_p():
  return pl.BlockSpec((2, _R, _D), lambda i: (0, i, 0))


def _spec_rows():
  return pl.BlockSpec((_R, _D), lambda i: (i, 0))


def _spec_w():
  return pl.BlockSpec((_D, _D), lambda i: (0, 0))


def _spec_b():
  return pl.BlockSpec((1, _D), lambda i: (0, 0))


def _spec_a():
  return pl.BlockSpec((1, 1), lambda i: (0, 0), memory_space=pltpu.SMEM)


_G = (_N // _R,)


_dense1 = pl.pallas_call(
    _dense1_body,
    grid=_G,
    in_specs=[_spec_p(), _spec_p(), _spec_rows(), _spec_w(), _spec_b(),
              _spec_w(), _spec_w(), _spec_b(), _spec_a()],
    out_specs=[_spec_rows(), _spec_rows()],
    out_shape=[jax.ShapeDtypeStruct((_N, _D), jnp.float32),
               jax.ShapeDtypeStruct((_N, _D), jnp.float32)],
)

_dense2 = pl.pallas_call(
    _dense2_body,
    grid=_G,
    in_specs=[_spec_p(), _spec_p(), _spec_rows(), _spec_w(), _spec_b(),
              _spec_w(), _spec_rows(), _spec_a()],
    out_specs=_spec_rows(),
    out_shape=jax.ShapeDtypeStruct((_N, _D), jnp.float32),
)

_dense3 = pl.pallas_call(
    _dense3_body,
    grid=_G,
    in_specs=[_spec_p(), _spec_p(), _spec_rows(), _spec_w(), _spec_b(),
              _spec_w(), _spec_a()],
    out_specs=_spec_rows(),
    out_shape=jax.ShapeDtypeStruct((_N, _D), jnp.float32),
)


def kernel(x, edge_index, Wskip, bskip, Wl1, bl1, Wr1,
           Wl2, bl2, Wr2, Wl3, bl3, Wr3, a):
  src = edge_index[0]
  dst = edge_index[1]
  pad = _EPAD - _E
  srcp = jnp.concatenate([src, jnp.zeros((pad,), jnp.int32)])
  dstp = jnp.concatenate([dst, jnp.full((pad,), _TRASH, jnp.int32)])
  idx = jnp.stack([srcp.reshape(-1, _CH), dstp.reshape(-1, _CH)], axis=1)
  zrows = jnp.zeros((_CH, _D), jnp.float32)
  ones = jnp.ones((_CH, _D), jnp.float32)
  a2 = a.reshape(1, 1)
  bl1r = bl1.reshape(1, _D)
  bl2r = bl2.reshape(1, _D)
  bl3r = bl3.reshape(1, _D)
  bskr = bskip.reshape(1, _D)

  cnt = _seg_cnt(idx, zrows, ones)
  p1 = _seg_sum(x, idx, zrows)
  g2, sk = _dense1(p1, cnt, x, Wl1, bl1r, Wr1, Wskip, bskr, a2)
  p2 = _seg_sum(g2, idx, zrows)
  g3 = _dense2(p2, cnt, g2, Wl2, bl2r, Wr2, sk, a2)
  p3 = _seg_sum(g3, idx, zrows)
  return _dense3(p3, cnt, g3, Wl3, bl3r, Wr3, a2)
